# baseline (speedup 1.0000x reference)
import functools

import jax
import jax.numpy as jnp
from jax.experimental import pallas as pl
from jax.experimental.pallas import tpu as pltpu

S_BLK = 256


def _attn_kernel(ctx_ref, q_ref, k_ref, v_ref, o_ref,
                 qs_ref, m_ref, l_ref, acc_ref, *,
                 scale, num_blocks, num_layers, kvh, g):
    b = pl.program_id(0)
    layer = pl.program_id(1)
    j = pl.program_id(2)
    ctx = ctx_ref[b]

    @pl.when(j == 0)
    def _init():
        m_ref[...] = jnp.full_like(m_ref, -1e30)
        l_ref[...] = jnp.zeros_like(l_ref)
        acc_ref[...] = jnp.zeros_like(acc_ref)

    @pl.when((j == 0) & (layer == 0))
    def _load_q():
        qs_ref[...] = q_ref[0] * scale

    @pl.when(j * S_BLK < ctx)
    def _compute():
        q = qs_ref[...]
        k = k_ref[0, 0]
        v = v_ref[0, 0]
        s = jax.lax.dot_general(
            q, k, (((2,), (2,)), ((0,), (0,))),
            preferred_element_type=jnp.float32)
        pos = j * S_BLK + jax.lax.broadcasted_iota(
            jnp.int32, (kvh, g, S_BLK), 2)
        s = jnp.where(pos < ctx, s, -1e30)

        m_prev = m_ref[...]
        s_max = jnp.max(s, axis=2, keepdims=True)
        m_new = jnp.maximum(m_prev, s_max)
        alpha = jnp.exp(m_prev - m_new)
        p = jnp.exp(s - m_new[:, :, :1])
        l_ref[...] = l_ref[...] * alpha + jnp.sum(p, axis=2, keepdims=True)
        acc_ref[...] = acc_ref[...] * alpha + jax.lax.dot_general(
            p, v, (((2,), (1,)), ((0,), (0,))),
            preferred_element_type=jnp.float32)
        m_ref[...] = m_new

    @pl.when(j == num_blocks - 1)
    def _finalize():
        out = acc_ref[...] / l_ref[...]

        @pl.when(layer == num_layers - 1)
        def _write_out():
            o_ref[0] = out

        @pl.when(layer < num_layers - 1)
        def _carry_q():
            qs_ref[...] = out * scale


@jax.jit
def kernel(query, k_cache, v_cache, context_lens):
    B, H, D = query.shape
    L = k_cache.shape[1]
    KVH = k_cache.shape[2]
    S = k_cache.shape[3]
    G = H // KVH
    scale = 1.0 / D ** 0.5
    num_blocks = S // S_BLK

    q4 = query.reshape(B, KVH, G, D)

    def q_map(b, layer, j, ctx):
        return (b, 0, 0, 0)

    def kv_map(b, layer, j, ctx):
        last = jax.lax.div(ctx[b] + (S_BLK - 1), S_BLK) - 1
        last = jnp.maximum(last, 0)
        return (b, layer, 0, jnp.minimum(j, last), 0)

    grid_spec = pltpu.PrefetchScalarGridSpec(
        num_scalar_prefetch=1,
        grid=(B, L, num_blocks),
        in_specs=[
            pl.BlockSpec((1, KVH, G, D), q_map),
            pl.BlockSpec((1, 1, KVH, S_BLK, D), kv_map),
            pl.BlockSpec((1, 1, KVH, S_BLK, D), kv_map),
        ],
        out_specs=pl.BlockSpec((1, KVH, G, D), q_map),
        scratch_shapes=[
            pltpu.VMEM((KVH, G, D), jnp.float32),
            pltpu.VMEM((KVH, G, 128), jnp.float32),
            pltpu.VMEM((KVH, G, 128), jnp.float32),
            pltpu.VMEM((KVH, G, D), jnp.float32),
        ],
    )
    out = pl.pallas_call(
        functools.partial(_attn_kernel, scale=scale, num_blocks=num_blocks,
                          num_layers=L, kvh=KVH, g=G),
        grid_spec=grid_spec,
        out_shape=jax.ShapeDtypeStruct((B, KVH, G, D), jnp.float32),
        compiler_params=pltpu.CompilerParams(
            dimension_semantics=("parallel", "arbitrary", "arbitrary"),
            vmem_limit_bytes=100 * 1024 * 1024),
    )(context_lens, q4, k_cache, v_cache)
    return out.reshape(B, H, D)


# speedup vs baseline: 1.7723x; 1.7723x over previous
import functools

import jax
import jax.numpy as jnp
from jax.experimental import pallas as pl
from jax.experimental.pallas import tpu as pltpu

S_BLK = 1024


def _attn_kernel(ctx_ref, q_ref, k_ref, v_ref, o_ref,
                 qs_ref, m_ref, l_ref, acc_ref, *,
                 scale, num_blocks, num_layers, kvh, g):
    b = pl.program_id(0)
    layer = pl.program_id(1)
    j = pl.program_id(2)
    ctx = ctx_ref[b]

    @pl.when(j == 0)
    def _init():
        m_ref[...] = jnp.full_like(m_ref, -1e30)
        l_ref[...] = jnp.zeros_like(l_ref)
        acc_ref[...] = jnp.zeros_like(acc_ref)

    @pl.when((j == 0) & (layer == 0))
    def _load_q():
        qs_ref[...] = q_ref[0] * scale

    @pl.when(j * S_BLK < ctx)
    def _compute():
        q = qs_ref[...]
        k = k_ref[0, 0]
        v = v_ref[0, 0]
        s = jax.lax.dot_general(
            q, k, (((2,), (2,)), ((0,), (0,))),
            preferred_element_type=jnp.float32)
        pos = j * S_BLK + jax.lax.broadcasted_iota(
            jnp.int32, (kvh, g, S_BLK), 2)
        s = jnp.where(pos < ctx, s, -1e30)

        m_prev = m_ref[...]
        s_max = jnp.max(s, axis=2, keepdims=True)
        m_new = jnp.maximum(m_prev, s_max)
        alpha = jnp.exp(m_prev - m_new)
        p = jnp.exp(s - m_new[:, :, :1])
        l_ref[...] = l_ref[...] * alpha + jnp.sum(p, axis=2, keepdims=True)
        acc_ref[...] = acc_ref[...] * alpha + jax.lax.dot_general(
            p, v, (((2,), (1,)), ((0,), (0,))),
            preferred_element_type=jnp.float32)
        m_ref[...] = m_new

    @pl.when(j == num_blocks - 1)
    def _finalize():
        out = acc_ref[...] / l_ref[...]

        @pl.when(layer == num_layers - 1)
        def _write_out():
            o_ref[0] = out

        @pl.when(layer < num_layers - 1)
        def _carry_q():
            qs_ref[...] = out * scale


@jax.jit
def kernel(query, k_cache, v_cache, context_lens):
    B, H, D = query.shape
    L = k_cache.shape[1]
    KVH = k_cache.shape[2]
    S = k_cache.shape[3]
    G = H // KVH
    scale = 1.0 / D ** 0.5
    num_blocks = S // S_BLK

    q4 = query.reshape(B, KVH, G, D)

    def q_map(b, layer, j, ctx):
        return (b, 0, 0, 0)

    def kv_map(b, layer, j, ctx):
        last = jax.lax.div(ctx[b] + (S_BLK - 1), S_BLK) - 1
        last = jnp.maximum(last, 0)
        return (b, layer, 0, jnp.minimum(j, last), 0)

    grid_spec = pltpu.PrefetchScalarGridSpec(
        num_scalar_prefetch=1,
        grid=(B, L, num_blocks),
        in_specs=[
            pl.BlockSpec((1, KVH, G, D), q_map),
            pl.BlockSpec((1, 1, KVH, S_BLK, D), kv_map),
            pl.BlockSpec((1, 1, KVH, S_BLK, D), kv_map),
        ],
        out_specs=pl.BlockSpec((1, KVH, G, D), q_map),
        scratch_shapes=[
            pltpu.VMEM((KVH, G, D), jnp.float32),
            pltpu.VMEM((KVH, G, 128), jnp.float32),
            pltpu.VMEM((KVH, G, 128), jnp.float32),
            pltpu.VMEM((KVH, G, D), jnp.float32),
        ],
    )
    out = pl.pallas_call(
        functools.partial(_attn_kernel, scale=scale, num_blocks=num_blocks,
                          num_layers=L, kvh=KVH, g=G),
        grid_spec=grid_spec,
        out_shape=jax.ShapeDtypeStruct((B, KVH, G, D), jnp.float32),
        compiler_params=pltpu.CompilerParams(
            dimension_semantics=("parallel", "arbitrary", "arbitrary"),
            vmem_limit_bytes=100 * 1024 * 1024),
    )(context_lens, q4, k_cache, v_cache)
    return out.reshape(B, H, D)
